# baseline (speedup 1.0000x reference)
import functools

import jax
import jax.numpy as jnp
from jax import lax
from jax.experimental import pallas as pl
from jax.experimental.pallas import tpu as pltpu
from jax.experimental.pallas import tpu_sc as plsc

_VOCAB = 100000
_D = 128
_BATCH = 4096
_HIST = 200
_B_TOTAL = _BATCH * _HIST
_NC, _NS = 2, 16
_NW = _NC * _NS
_CHUNK = 128
_B_PER_W = _B_TOTAL // _NW
_NCH = _B_PER_W // _CHUNK
_NBUF = 3


_mesh = plsc.VectorSubcoreMesh(
    core_axis_name="c", subcore_axis_name="s", num_cores=_NC, num_subcores=_NS
)


@functools.partial(
    pl.kernel,
    out_type=jax.ShapeDtypeStruct((_B_TOTAL, _D), jnp.float32),
    mesh=_mesh,
    scratch_types=[
        pltpu.VMEM((_NCH, _CHUNK), jnp.int32),
        [pltpu.VMEM((_CHUNK, _D), jnp.float32) for _ in range(_NBUF)],
        pltpu.VMEM_SHARED((_NS, _NBUF, _CHUNK, _D), jnp.float32),
        [pltpu.SemaphoreType.DMA for _ in range(_NBUF)],
        [pltpu.SemaphoreType.DMA for _ in range(_NBUF)],
        [pltpu.SemaphoreType.DMA for _ in range(_NBUF)],
    ],
)
def _emb_lookup(x_hbm, table_hbm, out_hbm, idx_v, rows, spm, gsems, hsems, wsems):
    sid = lax.axis_index("s")
    wid = sid * _NC + lax.axis_index("c")
    out0 = wid * _B_PER_W

    pltpu.sync_copy(x_hbm.at[pl.ds(wid * _NCH, _NCH)], idx_v)

    def fire_gather(b, ci):
        pltpu.async_copy(table_hbm.at[idx_v.at[ci]], rows[b], gsems[b])

    def wait_gather(b):
        pltpu.make_async_copy(out_hbm.at[pl.ds(0, _CHUNK)], rows[b], gsems[b]).wait()

    def fire_hop(b):
        pltpu.async_copy(rows[b], spm.at[sid, b], hsems[b])

    def wait_hop(b):
        pltpu.make_async_copy(rows[b], spm.at[sid, b], hsems[b]).wait()

    def fire_write(b, ci):
        pltpu.async_copy(
            spm.at[sid, b], out_hbm.at[pl.ds(out0 + ci * _CHUNK, _CHUNK)], wsems[b]
        )

    def wait_write(b):
        pltpu.make_async_copy(
            spm.at[sid, b], out_hbm.at[pl.ds(0, _CHUNK)], wsems[b]
        ).wait()

    def visit(ci, b, bp, bn, *, w_prev2_exists, has_next, has_prev):
        if w_prev2_exists:
            wait_write(bn)
        if has_next:
            fire_gather(bn, ci + 1)
        wait_gather(b)
        fire_hop(b)
        if has_prev:
            wait_hop(bp)
            fire_write(bp, ci - 1)

    fire_gather(0, 0)

    def super_body(s, carry):
        for v in range(_NBUF):
            ci = s * _NBUF + v
            b = v
            bp = (v - 1) % _NBUF
            bn = (v + 1) % _NBUF

            @pl.when(ci >= 2)
            def _():
                wait_write(bn)

            @pl.when(ci + 1 < _NCH)
            def _():
                fire_gather(bn, ci + 1)

            wait_gather(b)
            fire_hop(b)

            @pl.when(ci >= 1)
            def _():
                wait_hop(bp)
                fire_write(bp, ci - 1)

        return carry

    lax.fori_loop(0, (_NCH - 2) // _NBUF, super_body, 0)
    visit(198, 0, 2, 1, w_prev2_exists=True, has_next=True, has_prev=True)
    visit(199, 1, 0, 2, w_prev2_exists=True, has_next=False, has_prev=True)

    wait_hop(1)
    fire_write(1, _NCH - 1)
    wait_write(0)
    wait_write(1)


def kernel(x, table):
    xr = x.astype(jnp.int32).reshape(_B_TOTAL // _CHUNK, _CHUNK)
    out = _emb_lookup(xr, table)
    return out.reshape(_BATCH, _HIST, _D)
